# baseline (speedup 1.0000x reference)
import functools

import jax
import jax.numpy as jnp
from jax import lax
from jax.experimental import pallas as pl
from jax.experimental.pallas import tpu as pltpu
from jax.experimental.pallas import tpu_sc as plsc

TS = 20
N = 10000
E = 320000
F = 128
ALPHA = 0.2

NC = 2
NS = 16
BN = 1000

EPT1 = E // NS
EPT2 = E // (NC * NS)
CH1 = 400
CH2 = 80
NCH1 = EPT1 // CH1
NCH2 = EPT2 // CH2
NP1 = NCH1 // 2
NP2 = NCH2 // 2
RS = 624
RL = 640
DZ = RL // CH2


def _tc_body(x_ref, w_ref, a2_ref, h_ref, s8_ref):
    xb = x_ref[0]
    h = jnp.dot(xb, w_ref[0], preferred_element_type=jnp.float32)
    h_ref[0] = h
    s8_ref[0] = jnp.dot(h, a2_ref[0], preferred_element_type=jnp.float32)


def _tc_transform(x, W, A8):
    return pl.pallas_call(
        _tc_body,
        grid=(TS, N // BN),
        in_specs=[
            pl.BlockSpec((1, BN, F), lambda t, i: (t, i, 0)),
            pl.BlockSpec((1, F, F), lambda t, i: (t, 0, 0)),
            pl.BlockSpec((1, F, 8), lambda t, i: (t, 0, 0)),
        ],
        out_specs=[
            pl.BlockSpec((1, BN, F), lambda t, i: (t, i, 0)),
            pl.BlockSpec((1, BN, 8), lambda t, i: (t, i, 0)),
        ],
        out_shape=[
            jax.ShapeDtypeStruct((TS, N, F), jnp.float32),
            jax.ShapeDtypeStruct((TS, N, 8), jnp.float32),
        ],
    )(x, W, A8)


def _tc_combine_body(p_ref, o_ref):
    v = p_ref[0, 0] + p_ref[1, 0]
    o_ref[0] = jnp.where(v > 0.0, v, jnp.exp(v) - 1.0)


def _tc_combine(p):
    return pl.pallas_call(
        _tc_combine_body,
        grid=(TS, N // BN),
        in_specs=[pl.BlockSpec((2, 1, BN, F), lambda t, i: (0, t, i, 0))],
        out_specs=pl.BlockSpec((1, BN, F), lambda t, i: (t, i, 0)),
        out_shape=jax.ShapeDtypeStruct((TS, N, F), jnp.float32),
    )(p)


def _sc_gat(h_hbm, asrc_hbm, adst_hbm, src_hbm, dst_hbm, p_hbm, att_hbm,
            acc_sp, den_sp, asrc_sp, adst_sp,
            attsum_v,
            srcc1a, srcc1b, dstc1a, dstc1b, asc1a, asc1b, adc1a, adc1b,
            srcc2a, srcc2b, srcc2c, dstc2a, dstc2b, dstc2c,
            asc2a, asc2b, asc2c, adc2a, adc2b, adc2c,
            den2a, den2b, den2c, dsts2a, dsts2b, dsts2c,
            rows_a, rows_b, rows_c,
            zden_v, stage_v,
            ia0, ia1, ib0, ib1, ic0, ic1,
            ga0, ga1, ga2, ga3, gb0, gb1, gb2, gb3, gc0, gc1, gc2, gc3,
            sa0, sb0, sc0s):
    c = lax.axis_index("c")
    s = lax.axis_index("s")
    e1base = s * EPT1
    e2base = c * (E // NC) + s * EPT2
    rbase = s * RS
    z16 = jnp.zeros((16,), jnp.float32)

    srcc1 = (srcc1a, srcc1b)
    dstc1 = (dstc1a, dstc1b)
    asc1 = (asc1a, asc1b)
    adc1 = (adc1a, adc1b)
    srcc2 = (srcc2a, srcc2b, srcc2c)
    dstc2 = (dstc2a, dstc2b, dstc2c)
    asc2 = (asc2a, asc2b, asc2c)
    adc2 = (adc2a, adc2b, adc2c)
    den2 = (den2a, den2b, den2c)
    dsts2 = (dsts2a, dsts2b, dsts2c)
    rows = (rows_a, rows_b, rows_c)
    isem = ((ia0, ia1), (ib0, ib1), (ic0, ic1))
    gsem = ((ga0, ga1, ga2, ga3), (gb0, gb1, gb2, gb3),
            (gc0, gc1, gc2, gc3))
    ssem = (sa0, sb0, sc0s)

    def fi2(ci, b):
        base = e2base + ci * CH2
        return (
            pltpu.async_copy(src_hbm.at[pl.ds(base, CH2)], srcc2[b],
                             isem[b][0]),
            pltpu.async_copy(dst_hbm.at[pl.ds(base, CH2)], dstc2[b],
                             isem[b][1]),
        )

    def fg2(t, b):
        return (
            pltpu.async_copy(asrc_sp.at[srcc2[b]], asc2[b], gsem[b][0]),
            pltpu.async_copy(adst_sp.at[dstc2[b]], adc2[b], gsem[b][1]),
            pltpu.async_copy(den_sp.at[dstc2[b]], den2[b], gsem[b][2]),
            pltpu.async_copy(h_hbm.at[t].at[srcc2[b]], rows[b], gsem[b][3]),
        )

    def fs2(b):
        return pltpu.async_copy(rows[b], acc_sp.at[dsts2[b]], ssem[b],
                                add=True)

    def wall(ds):
        for d in ds:
            d.wait()

    def comp1(b):
        def k1(k, _):
            sl = pl.ds(k * 16, 16)
            e = asc1[b][sl] + adc1[b][sl]
            e = jnp.where(e >= 0.0, e, ALPHA * e)
            asc1[b][sl] = jnp.exp(e)
            return 0
        lax.fori_loop(0, CH1 // 16, k1, 0)

    def comp2(ci, b):
        def k2(k, _):
            sl = pl.ds(k * 16, 16)
            e = asc2[b][sl] + adc2[b][sl]
            e = jnp.where(e >= 0.0, e, ALPHA * e)
            w = jnp.exp(e)
            att16 = w / (den2[b][sl] + 1e-16)
            off = ci * CH2 + k * 16
            attsum_v[pl.ds(off, 16)] = attsum_v[pl.ds(off, 16)] + att16
            dsts2[b][sl] = dstc2[b][sl]
            for j in range(16):
                ab = jnp.broadcast_to(att16[j], (16,))
                er = k * 16 + j
                for q in range(F // 16):
                    rows[b][er, pl.ds(q * 16, 16)] = (
                        rows[b][er, pl.ds(q * 16, 16)] * ab)
            return 0
        lax.fori_loop(0, CH2 // 16, k2, 0)

    def _zd(i, _):
        zden_v[pl.ds(i * 16, 16)] = z16
        return 0
    lax.fori_loop(0, RL // 16, _zd, 0)

    def _za(i, _):
        attsum_v[pl.ds(i * 16, 16)] = z16
        return 0
    lax.fori_loop(0, EPT2 // 16, _za, 0)

    def ts_body(t, _):
        def zr(r, _):
            for q in range(F // 16):
                rows_a[r, pl.ds(q * 16, 16)] = z16
            return 0
        lax.fori_loop(0, CH2, zr, 0)
        for z in range(DZ):
            pltpu.sync_copy(rows_a, acc_sp.at[pl.ds(rbase + z * CH2, CH2)])
        pltpu.sync_copy(zden_v, den_sp.at[pl.ds(rbase, RL)])
        pltpu.sync_copy(asrc_hbm.at[pl.ds(t * N + rbase, RL)], stage_v)
        pltpu.sync_copy(stage_v, asrc_sp.at[pl.ds(rbase, RL)])
        pltpu.sync_copy(adst_hbm.at[pl.ds(t * N + rbase, RL)], stage_v)
        pltpu.sync_copy(stage_v, adst_sp.at[pl.ds(rbase, RL)])
        plsc.subcore_barrier()

        def p1pair(j, _):
            b0 = e1base + (2 * j) * CH1
            b1 = b0 + CH1
            dA1 = pltpu.async_copy(src_hbm.at[pl.ds(b0, CH1)], srcc1[0], ia0)
            dA2 = pltpu.async_copy(dst_hbm.at[pl.ds(b0, CH1)], dstc1[0], ia1)
            dB1 = pltpu.async_copy(src_hbm.at[pl.ds(b1, CH1)], srcc1[1], ib0)
            dB2 = pltpu.async_copy(dst_hbm.at[pl.ds(b1, CH1)], dstc1[1], ib1)
            dA1.wait()
            dA2.wait()
            gA1 = pltpu.async_copy(asrc_sp.at[srcc1[0]], asc1[0], ga0)
            gA2 = pltpu.async_copy(adst_sp.at[dstc1[0]], adc1[0], ga1)
            dB1.wait()
            dB2.wait()
            gB1 = pltpu.async_copy(asrc_sp.at[srcc1[1]], asc1[1], gb0)
            gB2 = pltpu.async_copy(adst_sp.at[dstc1[1]], adc1[1], gb1)
            gA1.wait()
            gA2.wait()
            comp1(0)
            sA = pltpu.async_copy(asc1[0], den_sp.at[dstc1[0]], sa0,
                                  add=True)
            gB1.wait()
            gB2.wait()
            comp1(1)
            sB = pltpu.async_copy(asc1[1], den_sp.at[dstc1[1]], sb0,
                                  add=True)
            sA.wait()
            sB.wait()
            return 0
        lax.fori_loop(0, NP1, p1pair, 0)
        plsc.subcore_barrier()

        def p2body(j, _):
            k0 = 5 * j
            i0 = fi2(k0, 0)
            i1 = fi2(k0 + 1, 1)
            i2 = fi2(k0 + 2, 2)
            wall(i0)
            g0 = fg2(t, 0)
            wall(i1)
            g1 = fg2(t, 1)
            wall(g0)
            comp2(k0, 0)
            s0 = fs2(0)
            wall(i2)
            g2 = fg2(t, 2)
            i3 = fi2(k0 + 3, 0)
            wall(g1)
            comp2(k0 + 1, 1)
            s1 = fs2(1)
            wall(i3)
            s0.wait()
            g3 = fg2(t, 0)
            i4 = fi2(k0 + 4, 1)
            wall(g2)
            comp2(k0 + 2, 2)
            s2 = fs2(2)
            wall(i4)
            s1.wait()
            g4 = fg2(t, 1)
            wall(g3)
            comp2(k0 + 3, 0)
            s3 = fs2(0)
            wall(g4)
            comp2(k0 + 4, 1)
            s4 = fs2(1)
            s2.wait()
            s3.wait()
            s4.wait()
            return 0
        lax.fori_loop(0, NCH2 // 5, p2body, 0)
        plsc.subcore_barrier()

        def dr(dj, _):
            r0 = rbase + (2 * dj) * CH2
            r1 = r0 + CH2
            l0 = pltpu.async_copy(acc_sp.at[pl.ds(r0, CH2)], rows_a, ga0)
            l1 = pltpu.async_copy(acc_sp.at[pl.ds(r1, CH2)], rows_b, gb0)
            l0.wait()
            st0 = pltpu.async_copy(rows_a, p_hbm.at[c, t, pl.ds(r0, CH2)],
                                   ga1)
            l1.wait()
            st1 = pltpu.async_copy(rows_b, p_hbm.at[c, t, pl.ds(r1, CH2)],
                                   gb1)
            st0.wait()
            st1.wait()
            return 0
        lax.fori_loop(0, DZ // 2, dr, 0)
        plsc.subcore_barrier()
        return 0

    lax.fori_loop(0, TS, ts_body, 0)

    def fin(i, _):
        attsum_v[pl.ds(i * 16, 16)] = attsum_v[pl.ds(i * 16, 16)] * (1.0 / TS)
        return 0
    lax.fori_loop(0, EPT2 // 16, fin, 0)
    pltpu.sync_copy(attsum_v, att_hbm.at[pl.ds(e2base, EPT2)])


_sc_gat_call = functools.partial(
    pl.kernel,
    out_type=[
        jax.ShapeDtypeStruct((NC, TS, N, F), jnp.float32),
        jax.ShapeDtypeStruct((E,), jnp.float32),
    ],
    mesh=plsc.VectorSubcoreMesh(
        core_axis_name="c", subcore_axis_name="s",
        num_cores=NC, num_subcores=NS),
    compiler_params=pltpu.CompilerParams(needs_layout_passes=False),
    scratch_types=(
        [
            pltpu.VMEM_SHARED((N, F), jnp.float32),
            pltpu.VMEM_SHARED((N,), jnp.float32),
            pltpu.VMEM_SHARED((N,), jnp.float32),
            pltpu.VMEM_SHARED((N,), jnp.float32),
            pltpu.VMEM((EPT2,), jnp.float32),
        ]
        + [pltpu.VMEM((CH1,), jnp.int32)] * 4
        + [pltpu.VMEM((CH1,), jnp.float32)] * 4
        + [pltpu.VMEM((CH2,), jnp.int32)] * 6
        + [pltpu.VMEM((CH2,), jnp.float32)] * 6
        + [pltpu.VMEM((CH2,), jnp.float32)] * 3
        + [pltpu.VMEM((CH2,), jnp.int32)] * 3
        + [pltpu.VMEM((CH2, F), jnp.float32)] * 3
        + [
            pltpu.VMEM((RL,), jnp.float32),
            pltpu.VMEM((RL,), jnp.float32),
        ]
        + [pltpu.SemaphoreType.DMA] * 21
    ),
)(_sc_gat)


def kernel(x, edge_index, W, a):
    A8 = jnp.zeros((TS, F, 8), jnp.float32)
    A8 = A8.at[:, :, 0].set(a[:, :F])
    A8 = A8.at[:, :, 1].set(a[:, F:])

    h_all, s8 = _tc_transform(x, W, A8)
    asrc_all = s8[:, :, 0].reshape(TS * N)
    adst_all = s8[:, :, 1].reshape(TS * N)

    src = edge_index[0]
    dst = edge_index[1]

    p, att_mean = _sc_gat_call(h_all, asrc_all, adst_all, src, dst)
    out = _tc_combine(p)
    return (out, att_mean)


# speedup vs baseline: 1.0055x; 1.0055x over previous
import functools

import jax
import jax.numpy as jnp
from jax import lax
from jax.experimental import pallas as pl
from jax.experimental.pallas import tpu as pltpu
from jax.experimental.pallas import tpu_sc as plsc

TS = 20
N = 10000
E = 320000
F = 128
ALPHA = 0.2

NC = 2
NS = 16
BN = 1000

EPT1 = E // NS
EPT2 = E // (NC * NS)
CH1 = 400
CH2 = 80
NCH1 = EPT1 // CH1
NCH2 = EPT2 // CH2
NP1 = NCH1 // 2
NP2 = NCH2 // 2
RS = 624
RL = 640
DZ = RL // CH2


def _tc_body(x_ref, w_ref, a2_ref, h_ref, s8_ref):
    xb = x_ref[0]
    h = jnp.dot(xb, w_ref[0], preferred_element_type=jnp.float32)
    h_ref[0] = h
    s8_ref[0] = jnp.dot(h, a2_ref[0], preferred_element_type=jnp.float32)


def _tc_transform(x, W, A8):
    return pl.pallas_call(
        _tc_body,
        grid=(TS, N // BN),
        in_specs=[
            pl.BlockSpec((1, BN, F), lambda t, i: (t, i, 0)),
            pl.BlockSpec((1, F, F), lambda t, i: (t, 0, 0)),
            pl.BlockSpec((1, F, 8), lambda t, i: (t, 0, 0)),
        ],
        out_specs=[
            pl.BlockSpec((1, BN, F), lambda t, i: (t, i, 0)),
            pl.BlockSpec((1, BN, 8), lambda t, i: (t, i, 0)),
        ],
        out_shape=[
            jax.ShapeDtypeStruct((TS, N, F), jnp.float32),
            jax.ShapeDtypeStruct((TS, N, 8), jnp.float32),
        ],
    )(x, W, A8)


def _tc_combine_body(p_ref, o_ref):
    v = p_ref[0, 0] + p_ref[1, 0]
    o_ref[0] = jnp.where(v > 0.0, v, jnp.exp(v) - 1.0)


def _tc_combine(p):
    return pl.pallas_call(
        _tc_combine_body,
        grid=(TS, N // BN),
        in_specs=[pl.BlockSpec((2, 1, BN, F), lambda t, i: (0, t, i, 0))],
        out_specs=pl.BlockSpec((1, BN, F), lambda t, i: (t, i, 0)),
        out_shape=jax.ShapeDtypeStruct((TS, N, F), jnp.float32),
    )(p)


def _sc_gat(h_hbm, asrc_hbm, adst_hbm, src_hbm, dst_hbm, p_hbm, att_hbm,
            acc_sp, den_sp, asrc_sp, adst_sp,
            attsum_v,
            srcc1a, srcc1b, dstc1a, dstc1b, asc1a, asc1b, adc1a, adc1b,
            srcc2a, srcc2b, srcc2c, dstc2a, dstc2b, dstc2c,
            asc2a, asc2b, asc2c, adc2a, adc2b, adc2c,
            den2a, den2b, den2c, dsts2a, dsts2b, dsts2c,
            rows_a, rows_b, rows_c,
            zden_v, stage_v,
            ia0, ia1, ib0, ib1, ic0, ic1,
            ga0, ga1, ga2, ga3, gb0, gb1, gb2, gb3, gc0, gc1, gc2, gc3,
            sa0, sb0, sc0s):
    c = lax.axis_index("c")
    s = lax.axis_index("s")
    e1base = s * EPT1
    e2base = c * (E // NC) + s * EPT2
    rbase = s * RS
    z16 = jnp.zeros((16,), jnp.float32)

    srcc1 = (srcc1a, srcc1b)
    dstc1 = (dstc1a, dstc1b)
    asc1 = (asc1a, asc1b)
    adc1 = (adc1a, adc1b)
    srcc2 = (srcc2a, srcc2b, srcc2c)
    dstc2 = (dstc2a, dstc2b, dstc2c)
    asc2 = (asc2a, asc2b, asc2c)
    adc2 = (adc2a, adc2b, adc2c)
    den2 = (den2a, den2b, den2c)
    dsts2 = (dsts2a, dsts2b, dsts2c)
    rows = (rows_a, rows_b, rows_c)
    isem = ((ia0, ia1), (ib0, ib1), (ic0, ic1))
    gsem = ((ga0, ga1, ga2, ga3), (gb0, gb1, gb2, gb3),
            (gc0, gc1, gc2, gc3))
    ssem = (sa0, sb0, sc0s)

    def fi2(ci, b):
        base = e2base + ci * CH2
        return (
            pltpu.async_copy(src_hbm.at[pl.ds(base, CH2)], srcc2[b],
                             isem[b][0]),
            pltpu.async_copy(dst_hbm.at[pl.ds(base, CH2)], dstc2[b],
                             isem[b][1]),
        )

    def fg2(t, b):
        return (
            pltpu.async_copy(asrc_sp.at[srcc2[b]], asc2[b], gsem[b][0]),
            pltpu.async_copy(adst_sp.at[dstc2[b]], adc2[b], gsem[b][1]),
            pltpu.async_copy(den_sp.at[dstc2[b]], den2[b], gsem[b][2]),
            pltpu.async_copy(h_hbm.at[t].at[srcc2[b]], rows[b], gsem[b][3]),
        )

    def fs2(b):
        return pltpu.async_copy(rows[b], acc_sp.at[dsts2[b]], ssem[b],
                                add=True)

    def wall(ds):
        for d in ds:
            d.wait()

    def comp1(b):
        def k1(k, _):
            sl = pl.ds(k * 16, 16)
            e = asc1[b][sl] + adc1[b][sl]
            e = jnp.where(e >= 0.0, e, ALPHA * e)
            asc1[b][sl] = jnp.exp(e)
            return 0
        lax.fori_loop(0, CH1 // 16, k1, 0)

    def comp2(ci, b):
        def k2(k, _):
            sl = pl.ds(k * 16, 16)
            e = asc2[b][sl] + adc2[b][sl]
            e = jnp.where(e >= 0.0, e, ALPHA * e)
            w = jnp.exp(e)
            att16 = w / (den2[b][sl] + 1e-16)
            off = ci * CH2 + k * 16
            attsum_v[pl.ds(off, 16)] = attsum_v[pl.ds(off, 16)] + att16
            dsts2[b][sl] = dstc2[b][sl]
            for j in range(16):
                ab = jnp.broadcast_to(att16[j], (16,))
                er = k * 16 + j
                for q in range(F // 16):
                    rows[b][er, pl.ds(q * 16, 16)] = (
                        rows[b][er, pl.ds(q * 16, 16)] * ab)
            return 0
        lax.fori_loop(0, CH2 // 16, k2, 0)

    def _zd(i, _):
        zden_v[pl.ds(i * 16, 16)] = z16
        return 0
    lax.fori_loop(0, RL // 16, _zd, 0)

    def _za(i, _):
        attsum_v[pl.ds(i * 16, 16)] = z16
        return 0
    lax.fori_loop(0, EPT2 // 16, _za, 0)

    def ts_body(t, _):
        def zr(r, _):
            for q in range(F // 16):
                rows_a[r, pl.ds(q * 16, 16)] = z16
            return 0
        lax.fori_loop(0, CH2, zr, 0)
        zsems = (ga0, ga1, ga2, ga3, gb0, gb1, gb2, gb3)
        zds = []
        for z in range(DZ):
            zds.append(pltpu.async_copy(
                rows_a, acc_sp.at[pl.ds(rbase + z * CH2, CH2)], zsems[z]))
        zdd = pltpu.async_copy(zden_v, den_sp.at[pl.ds(rbase, RL)], gc0)
        l1 = pltpu.async_copy(asrc_hbm.at[pl.ds(t * N + rbase, RL)], stage_v,
                              gc1)
        l1.wait()
        s1 = pltpu.async_copy(stage_v, asrc_sp.at[pl.ds(rbase, RL)], gc2)
        s1.wait()
        l2 = pltpu.async_copy(adst_hbm.at[pl.ds(t * N + rbase, RL)], stage_v,
                              gc3)
        l2.wait()
        s2 = pltpu.async_copy(stage_v, adst_sp.at[pl.ds(rbase, RL)], ia0)
        s2.wait()
        wall(zds)
        zdd.wait()
        plsc.subcore_barrier()

        def p1pair(j, _):
            b0 = e1base + (2 * j) * CH1
            b1 = b0 + CH1
            dA1 = pltpu.async_copy(src_hbm.at[pl.ds(b0, CH1)], srcc1[0], ia0)
            dA2 = pltpu.async_copy(dst_hbm.at[pl.ds(b0, CH1)], dstc1[0], ia1)
            dB1 = pltpu.async_copy(src_hbm.at[pl.ds(b1, CH1)], srcc1[1], ib0)
            dB2 = pltpu.async_copy(dst_hbm.at[pl.ds(b1, CH1)], dstc1[1], ib1)
            dA1.wait()
            dA2.wait()
            gA1 = pltpu.async_copy(asrc_sp.at[srcc1[0]], asc1[0], ga0)
            gA2 = pltpu.async_copy(adst_sp.at[dstc1[0]], adc1[0], ga1)
            dB1.wait()
            dB2.wait()
            gB1 = pltpu.async_copy(asrc_sp.at[srcc1[1]], asc1[1], gb0)
            gB2 = pltpu.async_copy(adst_sp.at[dstc1[1]], adc1[1], gb1)
            gA1.wait()
            gA2.wait()
            comp1(0)
            sA = pltpu.async_copy(asc1[0], den_sp.at[dstc1[0]], sa0,
                                  add=True)
            gB1.wait()
            gB2.wait()
            comp1(1)
            sB = pltpu.async_copy(asc1[1], den_sp.at[dstc1[1]], sb0,
                                  add=True)
            sA.wait()
            sB.wait()
            return 0
        lax.fori_loop(0, NP1, p1pair, 0)
        plsc.subcore_barrier()

        def p2body(j, _):
            k0 = 5 * j
            i0 = fi2(k0, 0)
            i1 = fi2(k0 + 1, 1)
            i2 = fi2(k0 + 2, 2)
            wall(i0)
            g0 = fg2(t, 0)
            wall(i1)
            g1 = fg2(t, 1)
            wall(g0)
            comp2(k0, 0)
            s0 = fs2(0)
            wall(i2)
            g2 = fg2(t, 2)
            i3 = fi2(k0 + 3, 0)
            wall(g1)
            comp2(k0 + 1, 1)
            s1 = fs2(1)
            wall(i3)
            s0.wait()
            g3 = fg2(t, 0)
            i4 = fi2(k0 + 4, 1)
            wall(g2)
            comp2(k0 + 2, 2)
            s2 = fs2(2)
            wall(i4)
            s1.wait()
            g4 = fg2(t, 1)
            wall(g3)
            comp2(k0 + 3, 0)
            s3 = fs2(0)
            wall(g4)
            comp2(k0 + 4, 1)
            s4 = fs2(1)
            s2.wait()
            s3.wait()
            s4.wait()
            return 0
        lax.fori_loop(0, NCH2 // 5, p2body, 0)
        plsc.subcore_barrier()

        def dr(dj, _):
            r0 = rbase + (2 * dj) * CH2
            r1 = r0 + CH2
            l0 = pltpu.async_copy(acc_sp.at[pl.ds(r0, CH2)], rows_a, ga0)
            l1 = pltpu.async_copy(acc_sp.at[pl.ds(r1, CH2)], rows_b, gb0)
            l0.wait()
            st0 = pltpu.async_copy(rows_a, p_hbm.at[c, t, pl.ds(r0, CH2)],
                                   ga1)
            l1.wait()
            st1 = pltpu.async_copy(rows_b, p_hbm.at[c, t, pl.ds(r1, CH2)],
                                   gb1)
            st0.wait()
            st1.wait()
            return 0
        lax.fori_loop(0, DZ // 2, dr, 0)
        plsc.subcore_barrier()
        return 0

    lax.fori_loop(0, TS, ts_body, 0)

    def fin(i, _):
        attsum_v[pl.ds(i * 16, 16)] = attsum_v[pl.ds(i * 16, 16)] * (1.0 / TS)
        return 0
    lax.fori_loop(0, EPT2 // 16, fin, 0)
    pltpu.sync_copy(attsum_v, att_hbm.at[pl.ds(e2base, EPT2)])


_sc_gat_call = functools.partial(
    pl.kernel,
    out_type=[
        jax.ShapeDtypeStruct((NC, TS, N, F), jnp.float32),
        jax.ShapeDtypeStruct((E,), jnp.float32),
    ],
    mesh=plsc.VectorSubcoreMesh(
        core_axis_name="c", subcore_axis_name="s",
        num_cores=NC, num_subcores=NS),
    compiler_params=pltpu.CompilerParams(needs_layout_passes=False),
    scratch_types=(
        [
            pltpu.VMEM_SHARED((N, F), jnp.float32),
            pltpu.VMEM_SHARED((N,), jnp.float32),
            pltpu.VMEM_SHARED((N,), jnp.float32),
            pltpu.VMEM_SHARED((N,), jnp.float32),
            pltpu.VMEM((EPT2,), jnp.float32),
        ]
        + [pltpu.VMEM((CH1,), jnp.int32)] * 4
        + [pltpu.VMEM((CH1,), jnp.float32)] * 4
        + [pltpu.VMEM((CH2,), jnp.int32)] * 6
        + [pltpu.VMEM((CH2,), jnp.float32)] * 6
        + [pltpu.VMEM((CH2,), jnp.float32)] * 3
        + [pltpu.VMEM((CH2,), jnp.int32)] * 3
        + [pltpu.VMEM((CH2, F), jnp.float32)] * 3
        + [
            pltpu.VMEM((RL,), jnp.float32),
            pltpu.VMEM((RL,), jnp.float32),
        ]
        + [pltpu.SemaphoreType.DMA] * 21
    ),
)(_sc_gat)


def kernel(x, edge_index, W, a):
    A8 = jnp.zeros((TS, F, 8), jnp.float32)
    A8 = A8.at[:, :, 0].set(a[:, :F])
    A8 = A8.at[:, :, 1].set(a[:, F:])

    h_all, s8 = _tc_transform(x, W, A8)
    asrc_all = s8[:, :, 0].reshape(TS * N)
    adst_all = s8[:, :, 1].reshape(TS * N)

    src = edge_index[0]
    dst = edge_index[1]

    p, att_mean = _sc_gat_call(h_all, asrc_all, adst_all, src, dst)
    out = _tc_combine(p)
    return (out, att_mean)


# speedup vs baseline: 1.0957x; 1.0897x over previous
import functools

import jax
import jax.numpy as jnp
from jax import lax
from jax.experimental import pallas as pl
from jax.experimental.pallas import tpu as pltpu
from jax.experimental.pallas import tpu_sc as plsc

TS = 20
N = 10000
E = 320000
F = 128
ALPHA = 0.2

NC = 2
NS = 16
BN = 1000

EPT1 = E // NS
EPT2 = E // (NC * NS)
CH1 = 400
CH2 = 80
NCH1 = EPT1 // CH1
NCH2 = EPT2 // CH2
NP1 = NCH1 // 2
NP2 = NCH2 // 2
RS = 624
RL = 640
DZ = RL // CH2


def _tc_body(x_ref, w_ref, a2_ref, h_ref, s8_ref):
    xb = x_ref[0]
    h = jnp.dot(xb, w_ref[0], preferred_element_type=jnp.float32)
    h_ref[0] = h
    s8_ref[0] = jnp.dot(h, a2_ref[0], preferred_element_type=jnp.float32)


def _tc_transform(x, W, A8):
    return pl.pallas_call(
        _tc_body,
        grid=(TS, N // BN),
        in_specs=[
            pl.BlockSpec((1, BN, F), lambda t, i: (t, i, 0)),
            pl.BlockSpec((1, F, F), lambda t, i: (t, 0, 0)),
            pl.BlockSpec((1, F, 8), lambda t, i: (t, 0, 0)),
        ],
        out_specs=[
            pl.BlockSpec((1, BN, F), lambda t, i: (t, i, 0)),
            pl.BlockSpec((1, BN, 8), lambda t, i: (t, i, 0)),
        ],
        out_shape=[
            jax.ShapeDtypeStruct((TS, N, F), jnp.float32),
            jax.ShapeDtypeStruct((TS, N, 8), jnp.float32),
        ],
    )(x, W, A8)


def _tc_combine_body(p_ref, o_ref):
    v = p_ref[0, 0] + p_ref[1, 0]
    o_ref[0] = jnp.where(v > 0.0, v, jnp.exp(v) - 1.0)


def _tc_combine(p):
    return pl.pallas_call(
        _tc_combine_body,
        grid=(TS, N // BN),
        in_specs=[pl.BlockSpec((2, 1, BN, F), lambda t, i: (0, t, i, 0))],
        out_specs=pl.BlockSpec((1, BN, F), lambda t, i: (t, i, 0)),
        out_shape=jax.ShapeDtypeStruct((TS, N, F), jnp.float32),
    )(p)


def _sc_gat(h_hbm, asrc_hbm, adst_hbm, src_hbm, dst_hbm, p_hbm, att_hbm,
            acc_sp, den_sp, asrc_sp, adst_sp,
            attsum_v,
            srcc1a, srcc1b, dstc1a, dstc1b, asc1a, asc1b, adc1a, adc1b,
            srcc2a, srcc2b, srcc2c, dstc2a, dstc2b, dstc2c,
            asc2a, asc2b, asc2c, adc2a, adc2b, adc2c,
            den2a, den2b, den2c, dsts2a, dsts2b, dsts2c,
            rows_a, rows_b, rows_c,
            zden_v, stage_v,
            ia0, ia1, ib0, ib1, ic0, ic1,
            ga0, ga1, ga2, ga3, gb0, gb1, gb2, gb3, gc0, gc1, gc2, gc3,
            sa0, sb0, sc0s):
    c = lax.axis_index("c")
    s = lax.axis_index("s")
    e1base = s * EPT1
    e2base = c * (E // NC) + s * EPT2
    rbase = s * RS
    z16 = jnp.zeros((16,), jnp.float32)

    srcc1 = (srcc1a, srcc1b)
    dstc1 = (dstc1a, dstc1b)
    asc1 = (asc1a, asc1b)
    adc1 = (adc1a, adc1b)
    srcc2 = (srcc2a, srcc2b, srcc2c)
    dstc2 = (dstc2a, dstc2b, dstc2c)
    asc2 = (asc2a, asc2b, asc2c)
    adc2 = (adc2a, adc2b, adc2c)
    den2 = (den2a, den2b, den2c)
    dsts2 = (dsts2a, dsts2b, dsts2c)
    rows = (rows_a, rows_b, rows_c)
    isem = ((ia0, ia1), (ib0, ib1), (ic0, ic1))
    gsem = ((ga0, ga1, ga2, ga3), (gb0, gb1, gb2, gb3),
            (gc0, gc1, gc2, gc3))
    ssem = (sa0, sb0, sc0s)

    def fi2(ci, b):
        base = e2base + ci * CH2
        return (
            pltpu.async_copy(src_hbm.at[pl.ds(base, CH2)], srcc2[b],
                             isem[b][0]),
            pltpu.async_copy(dst_hbm.at[pl.ds(base, CH2)], dstc2[b],
                             isem[b][1]),
        )

    def fg2(t, b):
        return (
            pltpu.async_copy(asrc_sp.at[srcc2[b]], asc2[b], gsem[b][0]),
            pltpu.async_copy(adst_sp.at[dstc2[b]], adc2[b], gsem[b][1]),
            pltpu.async_copy(den_sp.at[dstc2[b]], den2[b], gsem[b][2]),
            pltpu.async_copy(h_hbm.at[t].at[srcc2[b]], rows[b], gsem[b][3]),
        )

    def fs2(b):
        return pltpu.async_copy(rows[b], acc_sp.at[dsts2[b]], ssem[b],
                                add=True)

    def wall(ds):
        for d in ds:
            d.wait()

    def comp1(b):
        def k1(k, _):
            sl = pl.ds(k * 16, 16)
            e = asc1[b][sl] + adc1[b][sl]
            e = jnp.where(e >= 0.0, e, ALPHA * e)
            asc1[b][sl] = jnp.exp(e)
            return 0
        lax.fori_loop(0, CH1 // 16, k1, 0)

    def comp2(ci, b):
        def k2(k, _):
            sl = pl.ds(k * 16, 16)
            e = asc2[b][sl] + adc2[b][sl]
            e = jnp.where(e >= 0.0, e, ALPHA * e)
            w = jnp.exp(e)
            att16 = w / (den2[b][sl] + 1e-16)
            off = ci * CH2 + k * 16
            attsum_v[pl.ds(off, 16)] = attsum_v[pl.ds(off, 16)] + att16
            dsts2[b][sl] = dstc2[b][sl]
            for j in range(16):
                ab = jnp.broadcast_to(att16[j], (16,))
                er = k * 16 + j
                for q in range(F // 16):
                    rows[b][er, pl.ds(q * 16, 16)] = (
                        rows[b][er, pl.ds(q * 16, 16)] * ab)
            return 0
        lax.fori_loop(0, CH2 // 16, k2, 0)

    def _zd(i, _):
        zden_v[pl.ds(i * 16, 16)] = z16
        return 0
    lax.fori_loop(0, RL // 16, _zd, 0)

    def _za(i, _):
        attsum_v[pl.ds(i * 16, 16)] = z16
        return 0
    lax.fori_loop(0, EPT2 // 16, _za, 0)

    def ts_body(t, _):
        def zr(r, _):
            for q in range(F // 16):
                rows_a[r, pl.ds(q * 16, 16)] = z16
            return 0
        lax.fori_loop(0, CH2, zr, 0)
        zsems = (ga0, ga1, ga2, ga3, gb0, gb1, gb2, gb3)
        zds = []
        for z in range(DZ):
            zds.append(pltpu.async_copy(
                rows_a, acc_sp.at[pl.ds(rbase + z * CH2, CH2)], zsems[z]))
        zdd = pltpu.async_copy(zden_v, den_sp.at[pl.ds(rbase, RL)], gc0)
        l1 = pltpu.async_copy(asrc_hbm.at[pl.ds(t * N + rbase, RL)], stage_v,
                              gc1)
        l1.wait()
        s1 = pltpu.async_copy(stage_v, asrc_sp.at[pl.ds(rbase, RL)], gc2)
        s1.wait()
        l2 = pltpu.async_copy(adst_hbm.at[pl.ds(t * N + rbase, RL)], stage_v,
                              gc3)
        l2.wait()
        s2 = pltpu.async_copy(stage_v, adst_sp.at[pl.ds(rbase, RL)], ia0)
        s2.wait()
        wall(zds)
        zdd.wait()
        plsc.subcore_barrier()

        def p1pair(j, _):
            b0 = e1base + (2 * j) * CH1
            b1 = b0 + CH1
            dA1 = pltpu.async_copy(src_hbm.at[pl.ds(b0, CH1)], srcc1[0], ia0)
            dA2 = pltpu.async_copy(dst_hbm.at[pl.ds(b0, CH1)], dstc1[0], ia1)
            dB1 = pltpu.async_copy(src_hbm.at[pl.ds(b1, CH1)], srcc1[1], ib0)
            dB2 = pltpu.async_copy(dst_hbm.at[pl.ds(b1, CH1)], dstc1[1], ib1)
            dA1.wait()
            dA2.wait()
            gA1 = pltpu.async_copy(asrc_sp.at[srcc1[0]], asc1[0], ga0)
            gA2 = pltpu.async_copy(adst_sp.at[dstc1[0]], adc1[0], ga1)
            dB1.wait()
            dB2.wait()
            gB1 = pltpu.async_copy(asrc_sp.at[srcc1[1]], asc1[1], gb0)
            gB2 = pltpu.async_copy(adst_sp.at[dstc1[1]], adc1[1], gb1)
            gA1.wait()
            gA2.wait()
            comp1(0)
            sA = pltpu.async_copy(asc1[0], den_sp.at[dstc1[0]], sa0,
                                  add=True)
            gB1.wait()
            gB2.wait()
            comp1(1)
            sB = pltpu.async_copy(asc1[1], den_sp.at[dstc1[1]], sb0,
                                  add=True)
            sA.wait()
            sB.wait()
            return 0
        lax.fori_loop(0, NP1, p1pair, 0)
        plsc.subcore_barrier()

        def p2body(k0, n):
            i_d = [None, None, None]
            g_d = [None, None, None]
            s_d = [None, None, None]
            for m in range(min(3, n)):
                i_d[m] = fi2(k0 + m, m)
            wall(i_d[0])
            g_d[0] = fg2(t, 0)
            if n > 1:
                wall(i_d[1])
                g_d[1] = fg2(t, 1)
            for m in range(n):
                sl = m % 3
                wall(g_d[sl])
                comp2(k0 + m, sl)
                s_d[sl] = fs2(sl)
                if m + 3 < n:
                    i_d[sl] = fi2(k0 + m + 3, sl)
                nsl = (m + 2) % 3
                if m + 2 < n:
                    wall(i_d[nsl])
                    if s_d[nsl] is not None:
                        s_d[nsl].wait()
                        s_d[nsl] = None
                    g_d[nsl] = fg2(t, nsl)
            for sl in range(3):
                if s_d[sl] is not None:
                    s_d[sl].wait()

        def p2loop(j, _):
            p2body(10 * j, 10)
            return 0
        lax.fori_loop(0, NCH2 // 10, p2loop, 0)
        p2body((NCH2 // 10) * 10, NCH2 % 10)
        plsc.subcore_barrier()

        def dr(dj, _):
            r0 = rbase + (2 * dj) * CH2
            r1 = r0 + CH2
            l0 = pltpu.async_copy(acc_sp.at[pl.ds(r0, CH2)], rows_a, ga0)
            l1 = pltpu.async_copy(acc_sp.at[pl.ds(r1, CH2)], rows_b, gb0)
            l0.wait()
            st0 = pltpu.async_copy(rows_a, p_hbm.at[c, t, pl.ds(r0, CH2)],
                                   ga1)
            l1.wait()
            st1 = pltpu.async_copy(rows_b, p_hbm.at[c, t, pl.ds(r1, CH2)],
                                   gb1)
            st0.wait()
            st1.wait()
            return 0
        lax.fori_loop(0, DZ // 2, dr, 0)
        plsc.subcore_barrier()
        return 0

    lax.fori_loop(0, TS, ts_body, 0)

    def fin(i, _):
        attsum_v[pl.ds(i * 16, 16)] = attsum_v[pl.ds(i * 16, 16)] * (1.0 / TS)
        return 0
    lax.fori_loop(0, EPT2 // 16, fin, 0)
    pltpu.sync_copy(attsum_v, att_hbm.at[pl.ds(e2base, EPT2)])


_sc_gat_call = functools.partial(
    pl.kernel,
    out_type=[
        jax.ShapeDtypeStruct((NC, TS, N, F), jnp.float32),
        jax.ShapeDtypeStruct((E,), jnp.float32),
    ],
    mesh=plsc.VectorSubcoreMesh(
        core_axis_name="c", subcore_axis_name="s",
        num_cores=NC, num_subcores=NS),
    compiler_params=pltpu.CompilerParams(needs_layout_passes=False),
    scratch_types=(
        [
            pltpu.VMEM_SHARED((N, F), jnp.float32),
            pltpu.VMEM_SHARED((N,), jnp.float32),
            pltpu.VMEM_SHARED((N,), jnp.float32),
            pltpu.VMEM_SHARED((N,), jnp.float32),
            pltpu.VMEM((EPT2,), jnp.float32),
        ]
        + [pltpu.VMEM((CH1,), jnp.int32)] * 4
        + [pltpu.VMEM((CH1,), jnp.float32)] * 4
        + [pltpu.VMEM((CH2,), jnp.int32)] * 6
        + [pltpu.VMEM((CH2,), jnp.float32)] * 6
        + [pltpu.VMEM((CH2,), jnp.float32)] * 3
        + [pltpu.VMEM((CH2,), jnp.int32)] * 3
        + [pltpu.VMEM((CH2, F), jnp.float32)] * 3
        + [
            pltpu.VMEM((RL,), jnp.float32),
            pltpu.VMEM((RL,), jnp.float32),
        ]
        + [pltpu.SemaphoreType.DMA] * 21
    ),
)(_sc_gat)


def kernel(x, edge_index, W, a):
    A8 = jnp.zeros((TS, F, 8), jnp.float32)
    A8 = A8.at[:, :, 0].set(a[:, :F])
    A8 = A8.at[:, :, 1].set(a[:, F:])

    h_all, s8 = _tc_transform(x, W, A8)
    asrc_all = s8[:, :, 0].reshape(TS * N)
    adst_all = s8[:, :, 1].reshape(TS * N)

    src = edge_index[0]
    dst = edge_index[1]

    p, att_mean = _sc_gat_call(h_all, asrc_all, adst_all, src, dst)
    out = _tc_combine(p)
    return (out, att_mean)
